# baseline (speedup 1.0000x reference)
import functools

import jax
import jax.numpy as jnp
from jax import lax
from jax.experimental import pallas as pl
from jax.experimental.pallas import tpu as pltpu
from jax.experimental.pallas import tpu_sc as plsc

N_CORES = 2
N_SUBCORES = 16
N_WORKERS = N_CORES * N_SUBCORES
CHUNK = 128
N_BUF = 13


def kernel(x, table):
    batch, fields = x.shape
    _, d_embed = table.shape
    b_total = batch * fields
    n_per_w = b_total // N_WORKERS
    n_chunks = n_per_w // CHUNK

    idx = x.reshape(b_total)

    mesh = plsc.VectorSubcoreMesh(core_axis_name="c", subcore_axis_name="s")

    scratch = (
        [pltpu.VMEM((n_per_w,), jnp.int32)]
        + [pltpu.VMEM((CHUNK, d_embed), jnp.float32) for _ in range(N_BUF)]
        + [pltpu.SemaphoreType.DMA for _ in range(2 * N_BUF)]
    )

    @functools.partial(
        pl.kernel,
        mesh=mesh,
        out_type=jax.ShapeDtypeStruct((b_total, d_embed), jnp.float32),
        compiler_params=pltpu.CompilerParams(use_tc_tiling_on_sc=False),
        scratch_types=scratch,
    )
    def emb(table_hbm, idx_hbm, out_hbm, idx_v, *rest):
        bufs = rest[:N_BUF]
        gsems = rest[N_BUF:2 * N_BUF]
        osems = rest[2 * N_BUF:]

        wid = lax.axis_index("s") * N_CORES + lax.axis_index("c")
        base = wid * n_per_w
        pltpu.sync_copy(idx_hbm.at[pl.ds(base, n_per_w)], idx_v)

        def gather(c, s):
            return pltpu.make_async_copy(
                table_hbm.at[idx_v.at[pl.ds(c * CHUNK, CHUNK)]], bufs[s], gsems[s])

        def copyout(c, s):
            return pltpu.make_async_copy(
                bufs[s], out_hbm.at[pl.ds(base + c * CHUNK, CHUNK)], osems[s])

        for s in range(N_BUF):
            gather(s, s).start()
        for s in range(N_BUF):
            gather(s, s).wait()
            copyout(s, s).start()
        for s in range(N_BUF):
            copyout(s, s).wait()
            gather(N_BUF + s, s).start()
        for s in range(N_BUF):
            gather(N_BUF + s, s).wait()
            copyout(N_BUF + s, s).start()
        for s in range(N_BUF):
            copyout(N_BUF + s, s).wait()

    out = emb(table, idx)
    return out.reshape(batch, fields, d_embed)


# speedup vs baseline: 1.1103x; 1.1103x over previous
import functools

import jax
import jax.numpy as jnp
from jax import lax
from jax.experimental import pallas as pl
from jax.experimental.pallas import tpu as pltpu
from jax.experimental.pallas import tpu_sc as plsc

N_CORES = 2
N_SUBCORES = 16
N_WORKERS = N_CORES * N_SUBCORES
CHUNK = 128
N_BUF = 13


def kernel(x, table):
    batch, fields = x.shape
    n_rows, d_embed = table.shape
    b_total = batch * fields
    n_per_w = b_total // N_WORKERS
    n_chunks = n_per_w // CHUNK

    idx = x.T.reshape(b_total) * 2
    table2 = jnp.pad(table, ((0, 0), (0, d_embed))).reshape(2 * n_rows, d_embed)

    mesh = plsc.VectorSubcoreMesh(core_axis_name="c", subcore_axis_name="s")

    scratch = (
        [pltpu.VMEM((n_per_w,), jnp.int32)]
        + [pltpu.VMEM((CHUNK, d_embed), jnp.float32) for _ in range(N_BUF)]
        + [pltpu.SemaphoreType.DMA for _ in range(2 * N_BUF)]
    )

    @functools.partial(
        pl.kernel,
        mesh=mesh,
        out_type=jax.ShapeDtypeStruct((b_total, d_embed), jnp.float32),
        compiler_params=pltpu.CompilerParams(use_tc_tiling_on_sc=False),
        scratch_types=scratch,
    )
    def emb(table_hbm, idx_hbm, out_hbm, idx_v, *rest):
        bufs = rest[:N_BUF]
        gsems = rest[N_BUF:2 * N_BUF]
        osems = rest[2 * N_BUF:]

        wid = lax.axis_index("s") * N_CORES + lax.axis_index("c")
        base = wid * n_per_w
        pltpu.sync_copy(idx_hbm.at[pl.ds(base, n_per_w)], idx_v)

        def gather(c, s):
            return pltpu.make_async_copy(
                table_hbm.at[idx_v.at[pl.ds(c * CHUNK, CHUNK)]], bufs[s], gsems[s])

        def copyout(c, s):
            return pltpu.make_async_copy(
                bufs[s], out_hbm.at[pl.ds(base + c * CHUNK, CHUNK)], osems[s])

        for s in range(N_BUF):
            gather(s, s).start()
        for s in range(N_BUF):
            gather(s, s).wait()
            copyout(s, s).start()
        for s in range(N_BUF):
            copyout(s, s).wait()
            gather(N_BUF + s, s).start()
        for s in range(N_BUF):
            gather(N_BUF + s, s).wait()
            copyout(N_BUF + s, s).start()
        for s in range(N_BUF):
            copyout(N_BUF + s, s).wait()

    out = emb(table2, idx)
    return out.reshape(fields, batch, d_embed).transpose(1, 0, 2)
